# baseline (speedup 1.0000x reference)
import jax
import jax.numpy as jnp
from jax import lax
from jax.experimental import pallas as pl
from jax.experimental.pallas import tpu as pltpu
from jax.experimental.pallas import tpu_sc as plsc

BATCH = 16384
NCOLS = 5
DIM = 32

_info = plsc.get_sparse_core_info()
_NC, _NS, _L = _info.num_cores, _info.num_subcores, _info.num_lanes
_NW = _NC * _NS
_BPW = BATCH // _NW
_CH = 128
_NCH = _BPW // _CH


def _emb_body(cat, w0, w1, w2, w3, w4, out, block_v, idx_v, rows_v, sem,
              out_sem):
    tables = [w0, w1, w2, w3, w4]
    wid = lax.axis_index("s") * _NC + lax.axis_index("c")
    base = wid * _BPW
    pltpu.sync_copy(cat.at[pl.ds(base, _BPW), :], block_v)
    lane = lax.iota(jnp.int32, _L)
    for j in range(_BPW // _L):
        rows = lane + (j * _L)
        for t in range(NCOLS):
            v = plsc.load_gather(block_v, [rows, jnp.full((_L,), t, jnp.int32)])
            idx_v[t][pl.ds(j * _L, _L)] = v
    copies = []
    for t in range(NCOLS):
        for h in range(_NCH):
            copies.append(pltpu.async_copy(
                tables[t].at[idx_v[t].at[pl.ds(h * _CH, _CH)]],
                rows_v[t * _NCH + h], sem))
    outs = []
    for t in range(NCOLS):
        for h in range(_NCH):
            copies[t * _NCH + h].wait()
            outs.append(pltpu.async_copy(
                rows_v[t * _NCH + h],
                out.at[pl.ds(base + h * _CH, _CH), pl.ds(t * DIM, DIM)],
                out_sem))
    for o in outs:
        o.wait()


_emb = pl.kernel(
    _emb_body,
    mesh=plsc.VectorSubcoreMesh(core_axis_name="c", subcore_axis_name="s"),
    out_type=jax.ShapeDtypeStruct((BATCH, NCOLS * DIM), jnp.bfloat16),
    scratch_types=[
        pltpu.VMEM((_BPW, NCOLS), jnp.int32),
        [pltpu.VMEM((_BPW,), jnp.int32) for _ in range(NCOLS)],
        [pltpu.VMEM((_CH, DIM), jnp.bfloat16) for _ in range(NCOLS * _NCH)],
        pltpu.SemaphoreType.DMA,
        pltpu.SemaphoreType.DMA,
    ],
    compiler_params=pltpu.CompilerParams(use_tc_tiling_on_sc=False,
                                         needs_layout_passes=False),
)


def kernel(cat_tensor, W0, W1, W2, W3, W4):
    ws = [W.astype(jnp.bfloat16) for W in (W0, W1, W2, W3, W4)]
    return _emb(cat_tensor, *ws).astype(jnp.float32)


# speedup vs baseline: 1.1992x; 1.1992x over previous
import jax
import jax.numpy as jnp
from jax import lax
from jax.experimental import pallas as pl
from jax.experimental.pallas import tpu as pltpu
from jax.experimental.pallas import tpu_sc as plsc

BATCH = 16384
NCOLS = 5
DIM = 32

_info = plsc.get_sparse_core_info()
_NC, _NS, _L = _info.num_cores, _info.num_subcores, _info.num_lanes
_NW = _NC * _NS
_BPW = BATCH // _NW
_CH = 128
_NCH = _BPW // _CH


def _emb_body(cat, wall, out, block_v, idx_v, rows_v, sem,
              out_sem):
    tables = [wall.at[t] for t in range(NCOLS)]
    wid = lax.axis_index("s") * _NC + lax.axis_index("c")
    base = wid * _BPW
    pltpu.sync_copy(cat.at[pl.ds(base, _BPW), :], block_v)
    lane = lax.iota(jnp.int32, _L)
    for j in range(_BPW // _L):
        rows = lane + (j * _L)
        for t in range(NCOLS):
            v = plsc.load_gather(block_v, [rows, jnp.full((_L,), t, jnp.int32)])
            idx_v[t][pl.ds(j * _L, _L)] = v
    copies = []
    for t in range(NCOLS):
        for h in range(_NCH):
            copies.append(pltpu.async_copy(
                tables[t].at[idx_v[t].at[pl.ds(h * _CH, _CH)]],
                rows_v[t * _NCH + h], sem))
    outs = []
    for t in range(NCOLS):
        for h in range(_NCH):
            copies[t * _NCH + h].wait()
            outs.append(pltpu.async_copy(
                rows_v[t * _NCH + h],
                out.at[pl.ds(base + h * _CH, _CH), pl.ds(t * DIM, DIM)],
                out_sem))
    for o in outs:
        o.wait()


_emb = pl.kernel(
    _emb_body,
    mesh=plsc.VectorSubcoreMesh(core_axis_name="c", subcore_axis_name="s"),
    out_type=jax.ShapeDtypeStruct((BATCH, NCOLS * DIM), jnp.float32),
    scratch_types=[
        pltpu.VMEM((_BPW, NCOLS), jnp.int32),
        [pltpu.VMEM((_BPW,), jnp.int32) for _ in range(NCOLS)],
        [pltpu.VMEM((_CH, DIM), jnp.float32) for _ in range(NCOLS * _NCH)],
        pltpu.SemaphoreType.DMA,
        pltpu.SemaphoreType.DMA,
    ],
    compiler_params=pltpu.CompilerParams(use_tc_tiling_on_sc=False,
                                         needs_layout_passes=False),
)


def kernel(cat_tensor, W0, W1, W2, W3, W4):
    wall = jnp.stack([W0, W1, W2, W3, W4], axis=0)
    return _emb(cat_tensor, wall)


# speedup vs baseline: 1.5239x; 1.2708x over previous
import jax
import jax.numpy as jnp
from jax import lax
from jax.experimental import pallas as pl
from jax.experimental.pallas import tpu as pltpu
from jax.experimental.pallas import tpu_sc as plsc

BATCH = 16384
NCOLS = 5
DIM = 32

_info = plsc.get_sparse_core_info()
_NC, _NS = _info.num_cores, _info.num_subcores
_NW = _NC * _NS
_BPW = BATCH // _NW


def _emb_body(i0, i1, i2, i3, i4, w0, w1, w2, w3, w4, out,
              idx_v, rows_v, sem, out_sem):
    idxs = [i0, i1, i2, i3, i4]
    tables = [w0, w1, w2, w3, w4]
    wid = lax.axis_index("s") * _NC + lax.axis_index("c")
    base = wid * _BPW
    for t in range(NCOLS):
        pltpu.sync_copy(idxs[t].at[pl.ds(base, _BPW)], idx_v[t])
    copies = []
    for t in range(NCOLS):
        copies.append(
            pltpu.async_copy(tables[t].at[idx_v[t]], rows_v[t], sem))
    outs = []
    for t in range(NCOLS):
        copies[t].wait()
        outs.append(pltpu.async_copy(
            rows_v[t], out.at[pl.ds(base, _BPW), pl.ds(t * DIM, DIM)],
            out_sem))
    for t in range(NCOLS):
        outs[t].wait()


_emb = pl.kernel(
    _emb_body,
    mesh=plsc.VectorSubcoreMesh(core_axis_name="c", subcore_axis_name="s"),
    out_type=jax.ShapeDtypeStruct((BATCH, NCOLS * DIM), jnp.float32),
    scratch_types=[
        [pltpu.VMEM((_BPW,), jnp.int32) for _ in range(NCOLS)],
        [pltpu.VMEM((_BPW, DIM), jnp.float32) for _ in range(NCOLS)],
        pltpu.SemaphoreType.DMA,
        pltpu.SemaphoreType.DMA,
    ],
    compiler_params=pltpu.CompilerParams(use_tc_tiling_on_sc=False),
)


def kernel(cat_tensor, W0, W1, W2, W3, W4):
    cols = [cat_tensor[:, t] for t in range(NCOLS)]
    return _emb(*cols, W0, W1, W2, W3, W4)
